# Initial kernel scaffold; baseline (speedup 1.0000x reference)
import jax
import jax.numpy as jnp
from jax.experimental import pallas as pl


def kernel(x, W_enc, b_enc, W_dec):
    raise NotImplementedError("write your pallas kernel here")


# speedup vs baseline: 1.8451x; 1.8451x over previous
import functools

import jax
import jax.numpy as jnp
from jax.experimental import pallas as pl

INPUT_DIM = 2048
HIDDEN_DIM = 32768
K = 64
BATCH = 32

HB = 2048
N_BLK = HIDDEN_DIM // HB


def _enc_body(x_ref, w_ref, b_ref, f_ref):
    acc = jax.lax.dot_general(
        x_ref[...], w_ref[...],
        (((1,), (1,)), ((), ())),
        preferred_element_type=jnp.float32,
    )
    f_ref[...] = jnp.maximum(acc + b_ref[...], 0.0) + 0.0


def _topk_body(f_ref, t_ref, c_ref):
    fb = jax.lax.bitcast_convert_type(f_ref[...], jnp.int32)

    def val_step(i, t):
        shift = 30 - i
        cand = t | (jnp.int32(1) << shift)
        cnt = jnp.sum((fb >= cand).astype(jnp.int32), axis=1, keepdims=True)
        return jnp.where(cnt >= K, cand, t)

    t = jax.lax.fori_loop(0, 31, val_step, jnp.zeros((BATCH, 1), jnp.int32))

    cnt_gt = jnp.sum((fb > t).astype(jnp.int32), axis=1, keepdims=True)
    m = K - cnt_gt

    eq = (fb == t)
    idx = jax.lax.broadcasted_iota(jnp.int32, fb.shape, 1)

    def idx_step(i, c):
        shift = 15 - i
        cand = c | (jnp.int32(1) << shift)
        cnt = jnp.sum((eq & (idx < cand)).astype(jnp.int32), axis=1, keepdims=True)
        return jnp.where(cnt < m, cand, c)

    j = jax.lax.fori_loop(0, 16, idx_step, jnp.zeros((BATCH, 1), jnp.int32))
    j = jnp.where(m > 0, j, jnp.int32(-1))

    t_ref[...] = t
    c_ref[...] = j


def _dec_body(f_ref, t_ref, c_ref, w_ref, o_ref):
    i = pl.program_id(0)
    fb = jax.lax.bitcast_convert_type(f_ref[...], jnp.int32)
    t = t_ref[...]
    c = c_ref[...]
    idx = i * HB + jax.lax.broadcasted_iota(jnp.int32, fb.shape, 1)
    keep = (fb > t) | ((fb == t) & (idx <= c))
    vals = jnp.where(keep, f_ref[...], 0.0)
    part = jax.lax.dot_general(
        vals, w_ref[...],
        (((1,), (1,)), ((), ())),
        preferred_element_type=jnp.float32,
    )

    @pl.when(i == 0)
    def _():
        o_ref[...] = jnp.zeros_like(o_ref)

    o_ref[...] += part


@jax.jit
def kernel(x, W_enc, b_enc, W_dec):
    b2d = b_enc.reshape(1, HIDDEN_DIM)

    feats = pl.pallas_call(
        _enc_body,
        grid=(N_BLK,),
        in_specs=[
            pl.BlockSpec((BATCH, INPUT_DIM), lambda i: (0, 0)),
            pl.BlockSpec((HB, INPUT_DIM), lambda i: (i, 0)),
            pl.BlockSpec((1, HB), lambda i: (0, i)),
        ],
        out_specs=pl.BlockSpec((BATCH, HB), lambda i: (0, i)),
        out_shape=jax.ShapeDtypeStruct((BATCH, HIDDEN_DIM), jnp.float32),
    )(x, W_enc, b2d)

    tbits, cut = pl.pallas_call(
        _topk_body,
        in_specs=[pl.BlockSpec((BATCH, HIDDEN_DIM), lambda: (0, 0))],
        out_specs=[
            pl.BlockSpec((BATCH, 1), lambda: (0, 0)),
            pl.BlockSpec((BATCH, 1), lambda: (0, 0)),
        ],
        out_shape=[
            jax.ShapeDtypeStruct((BATCH, 1), jnp.int32),
            jax.ShapeDtypeStruct((BATCH, 1), jnp.int32),
        ],
    )(feats)

    recon = pl.pallas_call(
        _dec_body,
        grid=(N_BLK,),
        in_specs=[
            pl.BlockSpec((BATCH, HB), lambda i: (0, i)),
            pl.BlockSpec((BATCH, 1), lambda i: (0, 0)),
            pl.BlockSpec((BATCH, 1), lambda i: (0, 0)),
            pl.BlockSpec((INPUT_DIM, HB), lambda i: (0, i)),
        ],
        out_specs=pl.BlockSpec((BATCH, INPUT_DIM), lambda i: (0, 0)),
        out_shape=jax.ShapeDtypeStruct((BATCH, INPUT_DIM), jnp.float32),
    )(feats, tbits, cut, W_dec)

    return recon
